# baseline (speedup 1.0000x reference)
import functools

import jax
import jax.numpy as jnp
from jax import lax
from jax.experimental import pallas as pl
from jax.experimental.pallas import tpu as pltpu
from jax.experimental.pallas import tpu_sc as plsc

_LN_EPS = 1e-12
_NUM_WORKERS = 32
_GATHER_CHUNK = 32
_NBUF = 4


def _sc_gather(table, ids):
    n = ids.shape[0]
    h = table.shape[1]
    b_per_w = n // _NUM_WORKERS
    nchunk = b_per_w // _GATHER_CHUNK
    nbuf = min(_NBUF, nchunk)
    mesh = plsc.VectorSubcoreMesh(core_axis_name="c", subcore_axis_name="s")

    @functools.partial(
        pl.kernel,
        out_type=jax.ShapeDtypeStruct((n, h), table.dtype),
        mesh=mesh,
        scratch_types=(
            [pltpu.VMEM((b_per_w,), jnp.int32)]
            + [pltpu.VMEM((_GATHER_CHUNK, h), table.dtype) for _ in range(nbuf)]
            + [pltpu.SemaphoreType.DMA, pltpu.SemaphoreType.DMA]
        ),
    )
    def k(tab_hbm, idx_hbm, out_hbm, idx_v, *rest):
        bufs, (gsem, wsem) = rest[:nbuf], rest[nbuf:]
        wid = lax.axis_index("s") * 2 + lax.axis_index("c")
        base = wid * b_per_w
        pltpu.sync_copy(idx_hbm.at[pl.ds(base, b_per_w)], idx_v)

        gpend, wpend = [], []
        for j in range(nchunk):
            slot = j % nbuf
            if j >= nbuf:
                wpend.pop(0).wait()
            gpend.append(pltpu.async_copy(
                tab_hbm.at[idx_v.at[pl.ds(j * _GATHER_CHUNK, _GATHER_CHUNK)]],
                bufs[slot], gsem))
            if j >= 1:
                gpend.pop(0).wait()
                wpend.append(pltpu.async_copy(
                    bufs[(j - 1) % nbuf],
                    out_hbm.at[pl.ds(base + (j - 1) * _GATHER_CHUNK,
                                     _GATHER_CHUNK)],
                    wsem))
        gpend.pop(0).wait()
        wpend.append(pltpu.async_copy(
            bufs[(nchunk - 1) % nbuf],
            out_hbm.at[pl.ds(base + (nchunk - 1) * _GATHER_CHUNK,
                             _GATHER_CHUNK)],
            wsem))
        for w in wpend:
            w.wait()

    return k(table, ids)


def _tc_ln_body(g_ref, p_ref, t_ref, tab_ref, gam_ref, bet_ref, o_ref):
    tt = t_ref[0, 0, :].astype(jnp.float32)[:, None]
    base = tab_ref[0:1, :]
    diff = tab_ref[1:2, :] - tab_ref[0:1, :]
    x = g_ref[...] + p_ref[...] + base + tt * diff
    mean = jnp.mean(x, axis=1, keepdims=True)
    xc = x - mean
    var = jnp.mean(xc * xc, axis=1, keepdims=True)
    y = xc * lax.rsqrt(var + _LN_EPS)
    o_ref[...] = y * gam_ref[...] + bet_ref[...]


def _tc_ln(gathered, pos_emb, type_ids3, type_tab, gamma2, beta2, block):
    n, h = gathered.shape
    s = pos_emb.shape[0]
    b = n // s
    sb = s // block
    grid = (b, sb)
    return pl.pallas_call(
        _tc_ln_body,
        grid=grid,
        in_specs=[
            pl.BlockSpec((block, h), lambda i, j: (i * sb + j, 0)),
            pl.BlockSpec((block, h), lambda i, j: (j, 0)),
            pl.BlockSpec((1, 1, block), lambda i, j: (i, 0, j)),
            pl.BlockSpec((2, h), lambda i, j: (0, 0)),
            pl.BlockSpec((1, h), lambda i, j: (0, 0)),
            pl.BlockSpec((1, h), lambda i, j: (0, 0)),
        ],
        out_specs=pl.BlockSpec((block, h), lambda i, j: (i * sb + j, 0)),
        out_shape=jax.ShapeDtypeStruct((n, h), jnp.float32),
        compiler_params=pltpu.CompilerParams(
            dimension_semantics=("parallel", "parallel"),
        ),
    )(gathered, pos_emb, type_ids3, type_tab, gamma2, beta2)


_OVERLAP_CHUNKS = 4


def kernel(input_ids, token_type_ids, word_embeddings, position_embeddings,
           token_type_embeddings, ln_gamma, ln_beta):
    b, s = input_ids.shape
    h = word_embeddings.shape[1]
    ids = input_ids.astype(jnp.int32)
    tt = token_type_ids.astype(jnp.int32)
    gamma2 = ln_gamma.reshape(1, h)
    beta2 = ln_beta.reshape(1, h)
    c = _OVERLAP_CHUNKS if b % _OVERLAP_CHUNKS == 0 else 1
    rows_per = b // c
    outs = []
    for i in range(c):
        ids_i = ids[i * rows_per:(i + 1) * rows_per].reshape(-1)
        g_i = _sc_gather(word_embeddings, ids_i)
        tt_i = tt[i * rows_per:(i + 1) * rows_per].reshape(rows_per, 1, s)
        o_i = _tc_ln(g_i, position_embeddings[:s], tt_i,
                     token_type_embeddings, gamma2, beta2, block=256)
        outs.append(o_i.reshape(rows_per, s, h))
    return jnp.concatenate(outs, axis=0)


# speedup vs baseline: 1.2310x; 1.2310x over previous
import functools

import jax
import jax.numpy as jnp
from jax import lax
from jax.experimental import pallas as pl
from jax.experimental.pallas import tpu as pltpu
from jax.experimental.pallas import tpu_sc as plsc

_LN_EPS = 1e-12
_NUM_WORKERS = 32
_GATHER_CHUNK = 64
_NBUF = 2


def _sc_gather(table, ids):
    n = ids.shape[0]
    h = table.shape[1]
    b_per_w = n // _NUM_WORKERS
    nchunk = b_per_w // _GATHER_CHUNK
    nbuf = min(_NBUF, nchunk)
    mesh = plsc.VectorSubcoreMesh(core_axis_name="c", subcore_axis_name="s")

    @functools.partial(
        pl.kernel,
        out_type=jax.ShapeDtypeStruct((n, h), table.dtype),
        mesh=mesh,
        scratch_types=(
            [pltpu.VMEM((b_per_w,), jnp.int32)]
            + [pltpu.VMEM((_GATHER_CHUNK, h), table.dtype) for _ in range(nbuf)]
            + [pltpu.SemaphoreType.DMA, pltpu.SemaphoreType.DMA]
        ),
    )
    def k(tab_hbm, idx_hbm, out_hbm, idx_v, *rest):
        bufs, (gsem, wsem) = rest[:nbuf], rest[nbuf:]
        wid = lax.axis_index("s") * 2 + lax.axis_index("c")
        base = wid * b_per_w
        pltpu.sync_copy(idx_hbm.at[pl.ds(base, b_per_w)], idx_v)

        gpend, wpend = [], []
        for j in range(nchunk):
            slot = j % nbuf
            if j >= nbuf:
                wpend.pop(0).wait()
            gpend.append(pltpu.async_copy(
                tab_hbm.at[idx_v.at[pl.ds(j * _GATHER_CHUNK, _GATHER_CHUNK)]],
                bufs[slot], gsem))
            if j >= 1:
                gpend.pop(0).wait()
                wpend.append(pltpu.async_copy(
                    bufs[(j - 1) % nbuf],
                    out_hbm.at[pl.ds(base + (j - 1) * _GATHER_CHUNK,
                                     _GATHER_CHUNK)],
                    wsem))
        gpend.pop(0).wait()
        wpend.append(pltpu.async_copy(
            bufs[(nchunk - 1) % nbuf],
            out_hbm.at[pl.ds(base + (nchunk - 1) * _GATHER_CHUNK,
                             _GATHER_CHUNK)],
            wsem))
        for w in wpend:
            w.wait()

    return k(table, ids)


def _tc_ln_body(g_ref, p_ref, t_ref, tab_ref, gam_ref, bet_ref, o_ref):
    tt = t_ref[0, 0, :].astype(jnp.float32)[:, None]
    base = tab_ref[0:1, :]
    diff = tab_ref[1:2, :] - tab_ref[0:1, :]
    x = g_ref[...] + p_ref[...] + base + tt * diff
    mean = jnp.mean(x, axis=1, keepdims=True)
    xc = x - mean
    var = jnp.mean(xc * xc, axis=1, keepdims=True)
    y = xc * lax.rsqrt(var + _LN_EPS)
    o_ref[...] = y * gam_ref[...] + bet_ref[...]


def _tc_ln(gathered, pos_emb, type_ids3, type_tab, gamma2, beta2, block):
    n, h = gathered.shape
    s = pos_emb.shape[0]
    b = n // s
    sb = s // block
    grid = (b, sb)
    return pl.pallas_call(
        _tc_ln_body,
        grid=grid,
        in_specs=[
            pl.BlockSpec((block, h), lambda i, j: (i * sb + j, 0)),
            pl.BlockSpec((block, h), lambda i, j: (j, 0)),
            pl.BlockSpec((1, 1, block), lambda i, j: (i, 0, j)),
            pl.BlockSpec((2, h), lambda i, j: (0, 0)),
            pl.BlockSpec((1, h), lambda i, j: (0, 0)),
            pl.BlockSpec((1, h), lambda i, j: (0, 0)),
        ],
        out_specs=pl.BlockSpec((block, h), lambda i, j: (i * sb + j, 0)),
        out_shape=jax.ShapeDtypeStruct((n, h), jnp.float32),
        compiler_params=pltpu.CompilerParams(
            dimension_semantics=("parallel", "parallel"),
        ),
    )(gathered, pos_emb, type_ids3, type_tab, gamma2, beta2)


_OVERLAP_CHUNKS = 1


def kernel(input_ids, token_type_ids, word_embeddings, position_embeddings,
           token_type_embeddings, ln_gamma, ln_beta):
    b, s = input_ids.shape
    h = word_embeddings.shape[1]
    ids = input_ids.astype(jnp.int32)
    tt = token_type_ids.astype(jnp.int32)
    gamma2 = ln_gamma.reshape(1, h)
    beta2 = ln_beta.reshape(1, h)
    c = _OVERLAP_CHUNKS if b % _OVERLAP_CHUNKS == 0 else 1
    rows_per = b // c
    outs = []
    for i in range(c):
        ids_i = ids[i * rows_per:(i + 1) * rows_per].reshape(-1)
        g_i = _sc_gather(word_embeddings, ids_i)
        tt_i = tt[i * rows_per:(i + 1) * rows_per].reshape(rows_per, 1, s)
        o_i = _tc_ln(g_i, position_embeddings[:s], tt_i,
                     token_type_embeddings, gamma2, beta2, block=256)
        outs.append(o_i.reshape(rows_per, s, h))
    return jnp.concatenate(outs, axis=0)


# speedup vs baseline: 1.4356x; 1.1663x over previous
import functools

import jax
import jax.numpy as jnp
from jax import lax
from jax.experimental import pallas as pl
from jax.experimental.pallas import tpu as pltpu
from jax.experimental.pallas import tpu_sc as plsc

_LN_EPS = 1e-12
_NUM_WORKERS = 32
_GATHER_CHUNK = 64
_NBUF = 2


def _sc_gather(table, ids):
    n = ids.shape[0]
    h = table.shape[1]
    b_per_w = n // _NUM_WORKERS
    nchunk = b_per_w // _GATHER_CHUNK
    nbuf = min(_NBUF, nchunk)
    mesh = plsc.VectorSubcoreMesh(core_axis_name="c", subcore_axis_name="s")

    @functools.partial(
        pl.kernel,
        out_type=jax.ShapeDtypeStruct((n, h), table.dtype),
        mesh=mesh,
        scratch_types=(
            [pltpu.VMEM((b_per_w,), jnp.int32)]
            + [pltpu.VMEM((_GATHER_CHUNK, h), table.dtype) for _ in range(nbuf)]
            + [pltpu.SemaphoreType.DMA, pltpu.SemaphoreType.DMA]
        ),
    )
    def k(tab_hbm, idx_hbm, out_hbm, idx_v, *rest):
        bufs, (gsem, wsem) = rest[:nbuf], rest[nbuf:]
        wid = lax.axis_index("s") * 2 + lax.axis_index("c")
        base = wid * b_per_w
        pltpu.sync_copy(idx_hbm.at[pl.ds(base, b_per_w)], idx_v)

        gpend, wpend = [], []
        for j in range(nchunk):
            slot = j % nbuf
            if j >= nbuf:
                wpend.pop(0).wait()
            gpend.append(pltpu.async_copy(
                tab_hbm.at[idx_v.at[pl.ds(j * _GATHER_CHUNK, _GATHER_CHUNK)]],
                bufs[slot], gsem))
            if j >= 1:
                gpend.pop(0).wait()
                wpend.append(pltpu.async_copy(
                    bufs[(j - 1) % nbuf],
                    out_hbm.at[pl.ds(base + (j - 1) * _GATHER_CHUNK,
                                     _GATHER_CHUNK)],
                    wsem))
        gpend.pop(0).wait()
        wpend.append(pltpu.async_copy(
            bufs[(nchunk - 1) % nbuf],
            out_hbm.at[pl.ds(base + (nchunk - 1) * _GATHER_CHUNK,
                             _GATHER_CHUNK)],
            wsem))
        for w in wpend:
            w.wait()

    return k(table, ids)


def _ln_math(g_ref, p_ref, t_ref, tab_ref, gam_ref, bet_ref, o_ref):
    tt = t_ref[0, 0, :].astype(jnp.float32)[:, None]
    base = tab_ref[0:1, :]
    diff = tab_ref[1:2, :] - tab_ref[0:1, :]
    x = g_ref[...] + p_ref[...] + base + tt * diff
    mean = jnp.mean(x, axis=1, keepdims=True)
    xc = x - mean
    var = jnp.mean(xc * xc, axis=1, keepdims=True)
    y = xc * lax.rsqrt(var + _LN_EPS)
    o_ref[...] = y * gam_ref[...] + bet_ref[...]


def _tc_ln_body(g_ref, p_ref, t_ref, tab_ref, gam_ref, bet_ref, o_ref):
    _ln_math(g_ref, p_ref, t_ref, tab_ref, gam_ref, bet_ref, o_ref)


def _tc_ln_body_alias(d_ref, g_ref, p_ref, t_ref, tab_ref, gam_ref, bet_ref,
                      o_ref):
    del d_ref
    _ln_math(g_ref, p_ref, t_ref, tab_ref, gam_ref, bet_ref, o_ref)


def _tc_ln(dst, n_total, row0, gathered, pos_emb, type_ids3, type_tab,
           gamma2, beta2, block):
    nc, h = gathered.shape
    s = pos_emb.shape[0]
    bc = nc // s
    sb = s // block
    row0b = row0 // block
    data_specs = [
        pl.BlockSpec((block, h), lambda j, i: (i * sb + j, 0)),
        pl.BlockSpec((block, h), lambda j, i: (j, 0)),
        pl.BlockSpec((1, 1, block), lambda j, i: (i, 0, j)),
        pl.BlockSpec((2, h), lambda j, i: (0, 0)),
        pl.BlockSpec((1, h), lambda j, i: (0, 0)),
        pl.BlockSpec((1, h), lambda j, i: (0, 0)),
    ]
    out_spec = pl.BlockSpec((block, h), lambda j, i: (row0b + i * sb + j, 0))
    common = dict(
        grid=(sb, bc),
        out_specs=out_spec,
        out_shape=jax.ShapeDtypeStruct((n_total, h), jnp.float32),
        compiler_params=pltpu.CompilerParams(
            dimension_semantics=("parallel", "parallel"),
        ),
    )
    args = (gathered, pos_emb, type_ids3, type_tab, gamma2, beta2)
    if dst is None:
        return pl.pallas_call(_tc_ln_body, in_specs=data_specs, **common)(*args)
    return pl.pallas_call(
        _tc_ln_body_alias,
        in_specs=[pl.BlockSpec(memory_space=pl.ANY)] + data_specs,
        input_output_aliases={0: 0},
        **common,
    )(dst, *args)


_OVERLAP_CHUNKS = 2
_TC_BLOCK = 512


def kernel(input_ids, token_type_ids, word_embeddings, position_embeddings,
           token_type_embeddings, ln_gamma, ln_beta):
    b, s = input_ids.shape
    h = word_embeddings.shape[1]
    ids = input_ids.astype(jnp.int32)
    tt = token_type_ids.astype(jnp.int32)
    gamma2 = ln_gamma.reshape(1, h)
    beta2 = ln_beta.reshape(1, h)
    c = _OVERLAP_CHUNKS if b % _OVERLAP_CHUNKS == 0 else 1
    rows_per = b // c
    dst = None
    for i in range(c):
        ids_i = ids[i * rows_per:(i + 1) * rows_per].reshape(-1)
        g_i = _sc_gather(word_embeddings, ids_i)
        tt_i = tt[i * rows_per:(i + 1) * rows_per].reshape(rows_per, 1, s)
        dst = _tc_ln(dst, b * s, i * rows_per * s, g_i,
                     position_embeddings[:s], tt_i, token_type_embeddings,
                     gamma2, beta2, block=_TC_BLOCK)
    return dst.reshape(b, s, h)
